# baseline (speedup 1.0000x reference)
import jax
import jax.numpy as jnp
from jax.experimental import pallas as pl
from jax.experimental.pallas import tpu as pltpu

_FREQ_MASKS = 2
_TIME_MASKS = 5
_FREQ_WIDTH = 27
_TIME_WIDTH = 0.05


def _mask_starts(B, F, T, tw):
    k = jax.random.key(42)
    kf, kt = jax.random.split(k)
    f_starts = jax.random.randint(kf, (B, _FREQ_MASKS), 0, F - _FREQ_WIDTH + 1)
    t_starts = jax.random.randint(kt, (B, _TIME_MASKS), 0, T - tw + 1)
    return f_starts.astype(jnp.int32), t_starts.astype(jnp.int32)


def _body(fs_ref, ts_ref, x_ref, o_ref, *, F, T, tw):
    b = pl.program_id(0)
    fi = jax.lax.broadcasted_iota(jnp.int32, (F, 1), 0)
    ti = jax.lax.broadcasted_iota(jnp.int32, (1, T), 1)
    fm = jnp.zeros((F, 1), jnp.bool_)
    for j in range(_FREQ_MASKS):
        s = fs_ref[b, j]
        fm = fm | ((fi >= s) & (fi < s + _FREQ_WIDTH))
    tm = jnp.zeros((1, T), jnp.bool_)
    for j in range(_TIME_MASKS):
        s = ts_ref[b, j]
        tm = tm | ((ti >= s) & (ti < s + tw))
    mask = fm | tm
    o_ref[0] = jnp.where(mask, jnp.float32(0.0), x_ref[0])


def kernel(input_spec):
    B, F, T = input_spec.shape
    tw = int(_TIME_WIDTH * T)
    f_starts, t_starts = _mask_starts(B, F, T, tw)

    import functools
    body = functools.partial(_body, F=F, T=T, tw=tw)

    return pl.pallas_call(
        body,
        grid_spec=pltpu.PrefetchScalarGridSpec(
            num_scalar_prefetch=2,
            grid=(B,),
            in_specs=[pl.BlockSpec((1, F, T), lambda b, fs, ts: (b, 0, 0))],
            out_specs=pl.BlockSpec((1, F, T), lambda b, fs, ts: (b, 0, 0)),
        ),
        out_shape=jax.ShapeDtypeStruct((B, F, T), input_spec.dtype),
        compiler_params=pltpu.CompilerParams(
            dimension_semantics=("parallel",),
        ),
    )(f_starts, t_starts, input_spec)


# speedup vs baseline: 1.3378x; 1.3378x over previous
import functools

import jax
import jax.numpy as jnp
from jax.experimental import pallas as pl
from jax.experimental.pallas import tpu as pltpu

_FREQ_MASKS = 2
_TIME_MASKS = 5
_FREQ_WIDTH = 27
_TIME_WIDTH = 0.05

_SAMPLES_PER_BLOCK = 8


def _mask_starts(B, F, T, tw):
    k = jax.random.key(42)
    kf, kt = jax.random.split(k)
    f_starts = jax.random.randint(kf, (B, _FREQ_MASKS), 0, F - _FREQ_WIDTH + 1)
    t_starts = jax.random.randint(kt, (B, _TIME_MASKS), 0, T - tw + 1)
    return f_starts.astype(jnp.int32), t_starts.astype(jnp.int32)


def _body(fs_ref, ts_ref, x_ref, o_ref, *, F, T, tw):
    j = pl.program_id(0)
    fi = jax.lax.broadcasted_iota(jnp.int32, (F, 1), 0)
    ti = jax.lax.broadcasted_iota(jnp.int32, (1, T), 1)
    for s in range(_SAMPLES_PER_BLOCK):
        b = j * _SAMPLES_PER_BLOCK + s
        fm = jnp.zeros((F, 1), jnp.bool_)
        for m in range(_FREQ_MASKS):
            st = fs_ref[b, m]
            fm = fm | ((fi >= st) & (fi < st + _FREQ_WIDTH))
        tm = jnp.zeros((1, T), jnp.bool_)
        for m in range(_TIME_MASKS):
            st = ts_ref[b, m]
            tm = tm | ((ti >= st) & (ti < st + tw))
        rows = slice(s * F, (s + 1) * F)
        o_ref[rows, :] = jnp.where(fm | tm, jnp.float32(0.0), x_ref[rows, :])


def kernel(input_spec):
    B, F, T = input_spec.shape
    tw = int(_TIME_WIDTH * T)
    f_starts, t_starts = _mask_starts(B, F, T, tw)
    x2 = input_spec.reshape(B * F, T)
    block_rows = _SAMPLES_PER_BLOCK * F
    grid = (B // _SAMPLES_PER_BLOCK,)

    body = functools.partial(_body, F=F, T=T, tw=tw)
    out = pl.pallas_call(
        body,
        grid_spec=pltpu.PrefetchScalarGridSpec(
            num_scalar_prefetch=2,
            grid=grid,
            in_specs=[pl.BlockSpec((block_rows, T), lambda j, fs, ts: (j, 0))],
            out_specs=pl.BlockSpec((block_rows, T), lambda j, fs, ts: (j, 0)),
        ),
        out_shape=jax.ShapeDtypeStruct((B * F, T), input_spec.dtype),
        compiler_params=pltpu.CompilerParams(
            dimension_semantics=("arbitrary",),
        ),
    )(f_starts, t_starts, x2)
    return out.reshape(B, F, T)
